# baseline (speedup 1.0000x reference)
import functools

import jax
import jax.numpy as jnp
from jax import lax
from jax.experimental import pallas as pl
from jax.experimental.pallas import tpu as pltpu
from jax.experimental.pallas import tpu_sc as plsc

N = 10000
E = 320000
D = 128
NC, NS = 2, 16
NT = NC * NS
CHUNK = 128
EPT = 10240
EPAD = NT * EPT
NITER = EPT // CHUNK
NPAD = 10240
RPT = NPAD // NS
OPT = 624
HR = NPAD // D
BR = 1000


NBUF = 2
NHALF = NITER // 2
NGR = NHALF // NBUF


def _sc_segsum_body(h, srcp3, dstp3, zrows, out0, out1,
                    acc, sidx, didx, r0, r1, sg0, sg1):
    bufs = [r0, r1]
    sems = [sg0, sg1]
    c = lax.axis_index("c")
    s = lax.axis_index("s")
    tile = c * NS + s

    pltpu.sync_copy(zrows, acc.at[pl.ds(s * RPT, RPT)])
    plsc.subcore_barrier()

    for half in range(2):
        base = tile * NITER + half * NHALF
        pltpu.sync_copy(srcp3.at[pl.ds(base, NHALF)], sidx)
        pltpu.sync_copy(dstp3.at[pl.ds(base, NHALF)], didx)

        for b in range(NBUF):
            pltpu.async_copy(h.at[sidx.at[b, 0]], bufs[b], sems[b])

        def group(g, carry):
            for b in range(NBUF):
                i = g * NBUF + b
                pltpu.make_async_copy(h.at[sidx.at[i, 0]], bufs[b],
                                      sems[b]).wait()
                pltpu.sync_copy(bufs[b], acc.at[didx.at[i, 0]], add=True)

                @pl.when(g < NGR - 1)
                def _():
                    pltpu.async_copy(h.at[sidx.at[i + NBUF, 0]], bufs[b],
                                     sems[b])
            return carry

        lax.fori_loop(0, NGR, group, 0)
    plsc.subcore_barrier()

    rows = pl.ds(s * OPT, OPT)
    tail = pl.ds(NS * OPT, N - NS * OPT)

    @pl.when(c == 0)
    def _():
        pltpu.sync_copy(acc.at[rows], out0.at[rows])

    @pl.when(c == 1)
    def _():
        pltpu.sync_copy(acc.at[rows], out1.at[rows])

    @pl.when((c == 0) & (s == NS - 1))
    def _():
        pltpu.sync_copy(acc.at[tail], out0.at[tail])

    @pl.when((c == 1) & (s == NS - 1))
    def _():
        pltpu.sync_copy(acc.at[tail], out1.at[tail])


_sc_segsum = pl.kernel(
    _sc_segsum_body,
    out_type=(jax.ShapeDtypeStruct((N, D), jnp.float32),
              jax.ShapeDtypeStruct((N, D), jnp.float32)),
    mesh=plsc.VectorSubcoreMesh(core_axis_name="c", subcore_axis_name="s"),
    scratch_types=(
        [pltpu.VMEM_SHARED((NPAD, D), jnp.float32),
         pltpu.VMEM((NHALF, 1, CHUNK), jnp.int32),
         pltpu.VMEM((NHALF, 1, CHUNK), jnp.int32)]
        + [pltpu.VMEM((CHUNK, D), jnp.float32)] * NBUF
        + [pltpu.SemaphoreType.DMA] * NBUF
    ),
)


def _sc_degree_body(dstp, zrows, iota_hbm, out0, out1,
                    acc, hist, didx_v, idx80, sem):
    c = lax.axis_index("c")
    s = lax.axis_index("s")
    tile = c * NS + s

    @pl.when(s < HR // 8)
    def _():
        pltpu.sync_copy(zrows.at[pl.ds(0, 8)], acc.at[pl.ds(s * 8, 8)])

    pltpu.sync_copy(zrows.at[pl.ds(0, HR)], hist)
    pltpu.sync_copy(iota_hbm, idx80)
    plsc.subcore_barrier()

    def step(i, carry):
        pltpu.sync_copy(dstp.at[tile * NITER + i, 0], didx_v)
        for k in range(CHUNK // 16):
            d16 = didx_v[pl.ds(k * 16, 16)]
            cnt, last = plsc.scan_count(d16)
            plsc.addupdate_scatter(
                hist,
                [lax.shift_right_logical(d16, 7),
                 lax.bitwise_and(d16, 127)],
                cnt.astype(jnp.float32),
                mask=last,
            )
        return carry

    lax.fori_loop(0, NITER, step, 0)
    pltpu.sync_copy(hist, acc.at[idx80], add=True)
    plsc.subcore_barrier()

    @pl.when((c == 0) & (s < HR // 8))
    def _():
        pltpu.sync_copy(acc.at[pl.ds(s * 8, 8)], out0.at[pl.ds(s * 8, 8)])

    @pl.when((c == 1) & (s < HR // 8))
    def _():
        pltpu.sync_copy(acc.at[pl.ds(s * 8, 8)], out1.at[pl.ds(s * 8, 8)])


_sc_degree = pl.kernel(
    _sc_degree_body,
    out_type=(jax.ShapeDtypeStruct((HR, D), jnp.float32),
              jax.ShapeDtypeStruct((HR, D), jnp.float32)),
    mesh=plsc.VectorSubcoreMesh(core_axis_name="c", subcore_axis_name="s"),
    scratch_types=[
        pltpu.VMEM_SHARED((HR, D), jnp.float32),
        pltpu.VMEM((HR, D), jnp.float32),
        pltpu.VMEM((CHUNK,), jnp.int32),
        pltpu.VMEM((HR,), jnp.int32),
        pltpu.SemaphoreType.DMA,
    ],
    compiler_params=pltpu.CompilerParams(needs_layout_passes=False),
)


def _conv(h, mean, wl_ref, wr_ref, bl_ref, br_ref):
    return (jnp.dot(mean, wl_ref[:, :], preferred_element_type=jnp.float32)
            + jnp.dot(h, wr_ref[:, :], preferred_element_type=jnp.float32)
            + bl_ref[:, :] + br_ref[:, :])


def _tc_layer_body(residual, hp_ref, p0_ref, p1_ref, inv_ref, wl_ref, wr_ref,
                   bl_ref, br_ref, o_ref):
    h = hp_ref[:, :]
    mean = (p0_ref[:, :] + p1_ref[:, :]) * inv_ref[:, :]
    z = _conv(h, mean, wl_ref, wr_ref, bl_ref, br_ref)
    if residual:
        z = z + h
        mu = jnp.mean(z, axis=1, keepdims=True)
        var = jnp.mean((z - mu) ** 2, axis=1, keepdims=True)
        z = (z - mu) * lax.rsqrt(var + 1e-5)
    o_ref[:, :] = jnp.maximum(z, 0.0)


def _tc_last_body(hp_ref, p0_ref, p1_ref, inv_ref, wl_ref, wr_ref,
                  bl_ref, br_ref, woutp_ref, boutp_ref, o_ref):
    h = hp_ref[:, :]
    mean = (p0_ref[:, :] + p1_ref[:, :]) * inv_ref[:, :]
    z = _conv(h, mean, wl_ref, wr_ref, bl_ref, br_ref)
    z = z + h
    mu = jnp.mean(z, axis=1, keepdims=True)
    var = jnp.mean((z - mu) ** 2, axis=1, keepdims=True)
    z = (z - mu) * lax.rsqrt(var + 1e-5)
    z = jnp.maximum(z, 0.0)
    logits = jnp.dot(z, woutp_ref[:, :], preferred_element_type=jnp.float32)
    logits = logits + boutp_ref[:, :]
    o_ref[:, :] = logits[:, :2]


_ROW = lambda i: (i, 0)
_FIX = lambda i: (0, 0)


def _tc_layer(residual, h, p0, p1, inv_cnt, Wl, Wr, bl, br):
    return pl.pallas_call(
        functools.partial(_tc_layer_body, residual),
        grid=(N // BR,),
        in_specs=[
            pl.BlockSpec((BR, D), _ROW),
            pl.BlockSpec((BR, D), _ROW),
            pl.BlockSpec((BR, D), _ROW),
            pl.BlockSpec((BR, 1), _ROW),
            pl.BlockSpec((D, D), _FIX),
            pl.BlockSpec((D, D), _FIX),
            pl.BlockSpec((1, D), _FIX),
            pl.BlockSpec((1, D), _FIX),
        ],
        out_specs=pl.BlockSpec((BR, D), _ROW),
        out_shape=jax.ShapeDtypeStruct((N, D), jnp.float32),
    )(h, p0, p1, inv_cnt, Wl, Wr, bl.reshape(1, D), br.reshape(1, D))


def _tc_last(h, p0, p1, inv_cnt, Wl, Wr, bl, br, Woutp, boutp):
    return pl.pallas_call(
        _tc_last_body,
        grid=(N // BR,),
        in_specs=[
            pl.BlockSpec((BR, D), _ROW),
            pl.BlockSpec((BR, D), _ROW),
            pl.BlockSpec((BR, D), _ROW),
            pl.BlockSpec((BR, 1), _ROW),
            pl.BlockSpec((D, D), _FIX),
            pl.BlockSpec((D, D), _FIX),
            pl.BlockSpec((1, D), _FIX),
            pl.BlockSpec((1, D), _FIX),
            pl.BlockSpec((D, D), _FIX),
            pl.BlockSpec((1, D), _FIX),
        ],
        out_specs=pl.BlockSpec((BR, 2), _ROW),
        out_shape=jax.ShapeDtypeStruct((N, 2), jnp.float32),
    )(h, p0, p1, inv_cnt, Wl, Wr, bl.reshape(1, D), br.reshape(1, D),
      Woutp, boutp)


def kernel(x, edge_index, Wl0, bl0, Wr0, br0, Wl1, bl1, Wr1, br1,
           Wl2, bl2, Wr2, br2, Wout, bout):
    f32 = jnp.float32
    x = x.astype(f32)
    src = edge_index[0].astype(jnp.int32)
    dst = edge_index[1].astype(jnp.int32)
    srcp = jnp.concatenate(
        [src, jnp.zeros((EPAD - E,), jnp.int32)]).reshape(-1, 1, CHUNK)
    dummy = N + jnp.arange(EPAD - E, dtype=jnp.int32) % (NPAD - N)
    dstp = jnp.concatenate([dst, dummy]).reshape(-1, 1, CHUNK)
    zrows = jnp.zeros((RPT, D), f32)
    iota80 = jnp.arange(HR, dtype=jnp.int32)
    Woutp = jnp.zeros((D, D), f32).at[:, :2].set(Wout.astype(f32))
    boutp = jnp.zeros((1, D), f32).at[:, :2].set(bout.astype(f32)[None, :])

    c0, c1 = _sc_degree(dstp, zrows, iota80)
    cnt = (c0 + c1).reshape(NPAD)[:N].reshape(N, 1)
    inv_cnt = 1.0 / jnp.maximum(cnt, 1.0)

    h = x
    p0, p1 = _sc_segsum(h, srcp, dstp, zrows)
    h = _tc_layer(False, h, p0, p1, inv_cnt, Wl0, Wr0, bl0, br0)
    p0, p1 = _sc_segsum(h, srcp, dstp, zrows)
    h = _tc_layer(True, h, p0, p1, inv_cnt, Wl1, Wr1, bl1, br1)
    p0, p1 = _sc_segsum(h, srcp, dstp, zrows)
    return _tc_last(h, p0, p1, inv_cnt, Wl2, Wr2, bl2, br2, Woutp, boutp)


# speedup vs baseline: 1.0445x; 1.0445x over previous
import functools

import jax
import jax.numpy as jnp
from jax import lax
from jax.experimental import pallas as pl
from jax.experimental.pallas import tpu as pltpu
from jax.experimental.pallas import tpu_sc as plsc

N = 10000
E = 320000
D = 128
NC, NS = 2, 16
NT = NC * NS
CHUNK = 128
EPT = 10240
EPAD = NT * EPT
NITER = EPT // CHUNK
NPAD = 10240
RPT = NPAD // NS
OPT = 624
HR = NPAD // D
BR = 1000


NBUF = 2
NITER0 = 124
NITER1 = 36
NHALF0 = NITER0 // 2
NHALF1 = NITER1 // 2


def _run_edges(h, srcp3, dstp3, acc, sidx, didx, bufs, sems, base, niter):
    nhalf = niter // 2
    ngr = nhalf // NBUF
    for half in range(2):
        hbase = base + half * nhalf
        pltpu.sync_copy(srcp3.at[pl.ds(hbase, nhalf)],
                        sidx.at[pl.ds(0, nhalf)])
        pltpu.sync_copy(dstp3.at[pl.ds(hbase, nhalf)],
                        didx.at[pl.ds(0, nhalf)])

        for b in range(NBUF):
            pltpu.async_copy(h.at[sidx.at[b, 0]], bufs[b], sems[b])

        def group(g, carry):
            for b in range(NBUF):
                i = g * NBUF + b
                pltpu.make_async_copy(h.at[sidx.at[i, 0]], bufs[b],
                                      sems[b]).wait()
                pltpu.sync_copy(bufs[b], acc.at[didx.at[i, 0]], add=True)

                @pl.when(g < ngr - 1)
                def _():
                    pltpu.async_copy(h.at[sidx.at[i + NBUF, 0]], bufs[b],
                                     sems[b])
            return carry

        lax.fori_loop(0, ngr, group, 0)


def _sc_segsum_body(h, srcp3, dstp3, zrows, out0, out1,
                    acc, sidx, didx, r0, r1, sg0, sg1):
    bufs = [r0, r1]
    sems = [sg0, sg1]
    c = lax.axis_index("c")
    s = lax.axis_index("s")

    pltpu.sync_copy(zrows, acc.at[pl.ds(s * RPT, RPT)])
    plsc.subcore_barrier()

    @pl.when(c == 0)
    def _():
        _run_edges(h, srcp3, dstp3, acc, sidx, didx, bufs, sems,
                   s * NITER0, NITER0)

    @pl.when(c == 1)
    def _():
        _run_edges(h, srcp3, dstp3, acc, sidx, didx, bufs, sems,
                   NS * NITER0 + s * NITER1, NITER1)

    plsc.subcore_barrier()

    rows = pl.ds(s * OPT, OPT)
    tail = pl.ds(NS * OPT, N - NS * OPT)

    @pl.when(c == 0)
    def _():
        pltpu.sync_copy(acc.at[rows], out0.at[rows])

    @pl.when(c == 1)
    def _():
        pltpu.sync_copy(acc.at[rows], out1.at[rows])

    @pl.when((c == 0) & (s == NS - 1))
    def _():
        pltpu.sync_copy(acc.at[tail], out0.at[tail])

    @pl.when((c == 1) & (s == NS - 1))
    def _():
        pltpu.sync_copy(acc.at[tail], out1.at[tail])


_sc_segsum = pl.kernel(
    _sc_segsum_body,
    out_type=(jax.ShapeDtypeStruct((N, D), jnp.float32),
              jax.ShapeDtypeStruct((N, D), jnp.float32)),
    mesh=plsc.VectorSubcoreMesh(core_axis_name="c", subcore_axis_name="s"),
    scratch_types=(
        [pltpu.VMEM_SHARED((NPAD, D), jnp.float32),
         pltpu.VMEM((NHALF0, 1, CHUNK), jnp.int32),
         pltpu.VMEM((NHALF0, 1, CHUNK), jnp.int32)]
        + [pltpu.VMEM((CHUNK, D), jnp.float32)] * NBUF
        + [pltpu.SemaphoreType.DMA] * NBUF
    ),
)


def _sc_degree_body(dstp, zrows, iota_hbm, out0, out1,
                    acc, hist, didx_v, idx80, sem):
    c = lax.axis_index("c")
    s = lax.axis_index("s")
    tile = c * NS + s

    @pl.when(s < HR // 8)
    def _():
        pltpu.sync_copy(zrows.at[pl.ds(0, 8)], acc.at[pl.ds(s * 8, 8)])

    pltpu.sync_copy(zrows.at[pl.ds(0, HR)], hist)
    pltpu.sync_copy(iota_hbm, idx80)
    plsc.subcore_barrier()

    def step(i, carry):
        pltpu.sync_copy(dstp.at[tile * NITER + i, 0], didx_v)
        for k in range(CHUNK // 16):
            d16 = didx_v[pl.ds(k * 16, 16)]
            cnt, last = plsc.scan_count(d16)
            plsc.addupdate_scatter(
                hist,
                [lax.shift_right_logical(d16, 7),
                 lax.bitwise_and(d16, 127)],
                cnt.astype(jnp.float32),
                mask=last,
            )
        return carry

    lax.fori_loop(0, NITER, step, 0)
    pltpu.sync_copy(hist, acc.at[idx80], add=True)
    plsc.subcore_barrier()

    @pl.when((c == 0) & (s < HR // 8))
    def _():
        pltpu.sync_copy(acc.at[pl.ds(s * 8, 8)], out0.at[pl.ds(s * 8, 8)])

    @pl.when((c == 1) & (s < HR // 8))
    def _():
        pltpu.sync_copy(acc.at[pl.ds(s * 8, 8)], out1.at[pl.ds(s * 8, 8)])


_sc_degree = pl.kernel(
    _sc_degree_body,
    out_type=(jax.ShapeDtypeStruct((HR, D), jnp.float32),
              jax.ShapeDtypeStruct((HR, D), jnp.float32)),
    mesh=plsc.VectorSubcoreMesh(core_axis_name="c", subcore_axis_name="s"),
    scratch_types=[
        pltpu.VMEM_SHARED((HR, D), jnp.float32),
        pltpu.VMEM((HR, D), jnp.float32),
        pltpu.VMEM((CHUNK,), jnp.int32),
        pltpu.VMEM((HR,), jnp.int32),
        pltpu.SemaphoreType.DMA,
    ],
    compiler_params=pltpu.CompilerParams(needs_layout_passes=False),
)


def _conv(h, mean, wl_ref, wr_ref, bl_ref, br_ref):
    return (jnp.dot(mean, wl_ref[:, :], preferred_element_type=jnp.float32)
            + jnp.dot(h, wr_ref[:, :], preferred_element_type=jnp.float32)
            + bl_ref[:, :] + br_ref[:, :])


def _tc_layer_body(residual, hp_ref, p0_ref, p1_ref, inv_ref, wl_ref, wr_ref,
                   bl_ref, br_ref, o_ref):
    h = hp_ref[:, :]
    mean = (p0_ref[:, :] + p1_ref[:, :]) * inv_ref[:, :]
    z = _conv(h, mean, wl_ref, wr_ref, bl_ref, br_ref)
    if residual:
        z = z + h
        mu = jnp.mean(z, axis=1, keepdims=True)
        var = jnp.mean((z - mu) ** 2, axis=1, keepdims=True)
        z = (z - mu) * lax.rsqrt(var + 1e-5)
    o_ref[:, :] = jnp.maximum(z, 0.0)


def _tc_last_body(hp_ref, p0_ref, p1_ref, inv_ref, wl_ref, wr_ref,
                  bl_ref, br_ref, woutp_ref, boutp_ref, o_ref):
    h = hp_ref[:, :]
    mean = (p0_ref[:, :] + p1_ref[:, :]) * inv_ref[:, :]
    z = _conv(h, mean, wl_ref, wr_ref, bl_ref, br_ref)
    z = z + h
    mu = jnp.mean(z, axis=1, keepdims=True)
    var = jnp.mean((z - mu) ** 2, axis=1, keepdims=True)
    z = (z - mu) * lax.rsqrt(var + 1e-5)
    z = jnp.maximum(z, 0.0)
    logits = jnp.dot(z, woutp_ref[:, :], preferred_element_type=jnp.float32)
    logits = logits + boutp_ref[:, :]
    o_ref[:, :] = logits[:, :2]


_ROW = lambda i: (i, 0)
_FIX = lambda i: (0, 0)


def _tc_layer(residual, h, p0, p1, inv_cnt, Wl, Wr, bl, br):
    return pl.pallas_call(
        functools.partial(_tc_layer_body, residual),
        grid=(N // BR,),
        in_specs=[
            pl.BlockSpec((BR, D), _ROW),
            pl.BlockSpec((BR, D), _ROW),
            pl.BlockSpec((BR, D), _ROW),
            pl.BlockSpec((BR, 1), _ROW),
            pl.BlockSpec((D, D), _FIX),
            pl.BlockSpec((D, D), _FIX),
            pl.BlockSpec((1, D), _FIX),
            pl.BlockSpec((1, D), _FIX),
        ],
        out_specs=pl.BlockSpec((BR, D), _ROW),
        out_shape=jax.ShapeDtypeStruct((N, D), jnp.float32),
    )(h, p0, p1, inv_cnt, Wl, Wr, bl.reshape(1, D), br.reshape(1, D))


def _tc_last(h, p0, p1, inv_cnt, Wl, Wr, bl, br, Woutp, boutp):
    return pl.pallas_call(
        _tc_last_body,
        grid=(N // BR,),
        in_specs=[
            pl.BlockSpec((BR, D), _ROW),
            pl.BlockSpec((BR, D), _ROW),
            pl.BlockSpec((BR, D), _ROW),
            pl.BlockSpec((BR, 1), _ROW),
            pl.BlockSpec((D, D), _FIX),
            pl.BlockSpec((D, D), _FIX),
            pl.BlockSpec((1, D), _FIX),
            pl.BlockSpec((1, D), _FIX),
            pl.BlockSpec((D, D), _FIX),
            pl.BlockSpec((1, D), _FIX),
        ],
        out_specs=pl.BlockSpec((BR, 2), _ROW),
        out_shape=jax.ShapeDtypeStruct((N, 2), jnp.float32),
    )(h, p0, p1, inv_cnt, Wl, Wr, bl.reshape(1, D), br.reshape(1, D),
      Woutp, boutp)


def kernel(x, edge_index, Wl0, bl0, Wr0, br0, Wl1, bl1, Wr1, br1,
           Wl2, bl2, Wr2, br2, Wout, bout):
    f32 = jnp.float32
    x = x.astype(f32)
    src = edge_index[0].astype(jnp.int32)
    dst = edge_index[1].astype(jnp.int32)
    srcp = jnp.concatenate(
        [src, jnp.zeros((EPAD - E,), jnp.int32)]).reshape(-1, 1, CHUNK)
    dummy = N + jnp.arange(EPAD - E, dtype=jnp.int32) % (NPAD - N)
    dstp = jnp.concatenate([dst, dummy]).reshape(-1, 1, CHUNK)
    zrows = jnp.zeros((RPT, D), f32)
    iota80 = jnp.arange(HR, dtype=jnp.int32)
    Woutp = jnp.zeros((D, D), f32).at[:, :2].set(Wout.astype(f32))
    boutp = jnp.zeros((1, D), f32).at[:, :2].set(bout.astype(f32)[None, :])

    c0, c1 = _sc_degree(dstp, zrows, iota80)
    cnt = (c0 + c1).reshape(NPAD)[:N].reshape(N, 1)
    inv_cnt = 1.0 / jnp.maximum(cnt, 1.0)

    h = x
    p0, p1 = _sc_segsum(h, srcp, dstp, zrows)
    h = _tc_layer(False, h, p0, p1, inv_cnt, Wl0, Wr0, bl0, br0)
    p0, p1 = _sc_segsum(h, srcp, dstp, zrows)
    h = _tc_layer(True, h, p0, p1, inv_cnt, Wl1, Wr1, bl1, br1)
    p0, p1 = _sc_segsum(h, srcp, dstp, zrows)
    return _tc_last(h, p0, p1, inv_cnt, Wl2, Wr2, bl2, br2, Woutp, boutp)


# speedup vs baseline: 1.0448x; 1.0003x over previous
import functools

import jax
import jax.numpy as jnp
from jax import lax
from jax.experimental import pallas as pl
from jax.experimental.pallas import tpu as pltpu
from jax.experimental.pallas import tpu_sc as plsc

N = 10000
E = 320000
D = 128
NC, NS = 2, 16
NT = NC * NS
CHUNK = 128
EPT = 10240
EPAD = NT * EPT
NITER = EPT // CHUNK
NPAD = 10240
RPT = NPAD // NS
OPT = 624
HR = NPAD // D
BR = 1000


NBUF = 2
NITER0 = 124
NITER1 = 36
NHALF0 = NITER0 // 2
NHALF1 = NITER1 // 2


def _run_edges(h, srcp3, dstp3, acc, sidx, didx, bufs, sems, base, niter):
    nhalf = niter // 2
    ngr = nhalf // NBUF
    for half in range(2):
        hbase = base + half * nhalf
        pltpu.sync_copy(srcp3.at[pl.ds(hbase, nhalf)],
                        sidx.at[pl.ds(0, nhalf)])
        pltpu.sync_copy(dstp3.at[pl.ds(hbase, nhalf)],
                        didx.at[pl.ds(0, nhalf)])

        for b in range(NBUF):
            pltpu.async_copy(h.at[sidx.at[b, 0]], bufs[b], sems[b])

        def group(g, carry):
            for b in range(NBUF):
                i = g * NBUF + b
                pltpu.make_async_copy(h.at[sidx.at[i, 0]], bufs[b],
                                      sems[b]).wait()
                pltpu.sync_copy(bufs[b], acc.at[didx.at[i, 0]], add=True)

                @pl.when(g < ngr - 1)
                def _():
                    pltpu.async_copy(h.at[sidx.at[i + NBUF, 0]], bufs[b],
                                     sems[b])
            return carry

        lax.fori_loop(0, ngr, group, 0)


def _sc_segsum_body(h, srcp3, dstp3, zrows, out0, out1,
                    acc, sidx, didx, r0, r1, sg0, sg1):
    bufs = [r0, r1]
    sems = [sg0, sg1]
    c = lax.axis_index("c")
    s = lax.axis_index("s")

    with jax.named_scope("zero"):
        pltpu.sync_copy(zrows, acc.at[pl.ds(s * RPT, RPT)])
        plsc.subcore_barrier()

    with jax.named_scope("edges"):
        @pl.when(c == 0)
        def _():
            _run_edges(h, srcp3, dstp3, acc, sidx, didx, bufs, sems,
                       s * NITER0, NITER0)

        @pl.when(c == 1)
        def _():
            _run_edges(h, srcp3, dstp3, acc, sidx, didx, bufs, sems,
                       NS * NITER0 + s * NITER1, NITER1)

    with jax.named_scope("waitall"):
        plsc.subcore_barrier()

    rows = pl.ds(s * OPT, OPT)
    tail = pl.ds(NS * OPT, N - NS * OPT)

    with jax.named_scope("writeout"):
        @pl.when(c == 0)
        def _():
            pltpu.sync_copy(acc.at[rows], out0.at[rows])

        @pl.when(c == 1)
        def _():
            pltpu.sync_copy(acc.at[rows], out1.at[rows])

        @pl.when((c == 0) & (s == NS - 1))
        def _():
            pltpu.sync_copy(acc.at[tail], out0.at[tail])

        @pl.when((c == 1) & (s == NS - 1))
        def _():
            pltpu.sync_copy(acc.at[tail], out1.at[tail])


_sc_segsum = pl.kernel(
    _sc_segsum_body,
    out_type=(jax.ShapeDtypeStruct((N, D), jnp.float32),
              jax.ShapeDtypeStruct((N, D), jnp.float32)),
    mesh=plsc.VectorSubcoreMesh(core_axis_name="c", subcore_axis_name="s"),
    scratch_types=(
        [pltpu.VMEM_SHARED((NPAD, D), jnp.float32),
         pltpu.VMEM((NHALF0, 1, CHUNK), jnp.int32),
         pltpu.VMEM((NHALF0, 1, CHUNK), jnp.int32)]
        + [pltpu.VMEM((CHUNK, D), jnp.float32)] * NBUF
        + [pltpu.SemaphoreType.DMA] * NBUF
    ),
)


def _sc_degree_body(dstp, zrows, iota_hbm, out0, out1,
                    acc, hist, didx_v, idx80, sem):
    c = lax.axis_index("c")
    s = lax.axis_index("s")
    tile = c * NS + s

    @pl.when(s < HR // 8)
    def _():
        pltpu.sync_copy(zrows.at[pl.ds(0, 8)], acc.at[pl.ds(s * 8, 8)])

    pltpu.sync_copy(zrows.at[pl.ds(0, HR)], hist)
    pltpu.sync_copy(iota_hbm, idx80)
    plsc.subcore_barrier()

    def step(i, carry):
        pltpu.sync_copy(dstp.at[tile * NITER + i, 0], didx_v)
        for k in range(CHUNK // 16):
            d16 = didx_v[pl.ds(k * 16, 16)]
            cnt, last = plsc.scan_count(d16)
            plsc.addupdate_scatter(
                hist,
                [lax.shift_right_logical(d16, 7),
                 lax.bitwise_and(d16, 127)],
                cnt.astype(jnp.float32),
                mask=last,
            )
        return carry

    lax.fori_loop(0, NITER, step, 0)
    pltpu.sync_copy(hist, acc.at[idx80], add=True)
    plsc.subcore_barrier()

    @pl.when((c == 0) & (s < HR // 8))
    def _():
        pltpu.sync_copy(acc.at[pl.ds(s * 8, 8)], out0.at[pl.ds(s * 8, 8)])

    @pl.when((c == 1) & (s < HR // 8))
    def _():
        pltpu.sync_copy(acc.at[pl.ds(s * 8, 8)], out1.at[pl.ds(s * 8, 8)])


_sc_degree = pl.kernel(
    _sc_degree_body,
    out_type=(jax.ShapeDtypeStruct((HR, D), jnp.float32),
              jax.ShapeDtypeStruct((HR, D), jnp.float32)),
    mesh=plsc.VectorSubcoreMesh(core_axis_name="c", subcore_axis_name="s"),
    scratch_types=[
        pltpu.VMEM_SHARED((HR, D), jnp.float32),
        pltpu.VMEM((HR, D), jnp.float32),
        pltpu.VMEM((CHUNK,), jnp.int32),
        pltpu.VMEM((HR,), jnp.int32),
        pltpu.SemaphoreType.DMA,
    ],
    compiler_params=pltpu.CompilerParams(needs_layout_passes=False),
)


def _conv(h, mean, wl_ref, wr_ref, bl_ref, br_ref):
    return (jnp.dot(mean, wl_ref[:, :], preferred_element_type=jnp.float32)
            + jnp.dot(h, wr_ref[:, :], preferred_element_type=jnp.float32)
            + bl_ref[:, :] + br_ref[:, :])


def _tc_layer_body(residual, hp_ref, p0_ref, p1_ref, inv_ref, wl_ref, wr_ref,
                   bl_ref, br_ref, o_ref):
    h = hp_ref[:, :]
    mean = (p0_ref[:, :] + p1_ref[:, :]) * inv_ref[:, :]
    z = _conv(h, mean, wl_ref, wr_ref, bl_ref, br_ref)
    if residual:
        z = z + h
        mu = jnp.mean(z, axis=1, keepdims=True)
        var = jnp.mean((z - mu) ** 2, axis=1, keepdims=True)
        z = (z - mu) * lax.rsqrt(var + 1e-5)
    o_ref[:, :] = jnp.maximum(z, 0.0)


def _tc_last_body(hp_ref, p0_ref, p1_ref, inv_ref, wl_ref, wr_ref,
                  bl_ref, br_ref, woutp_ref, boutp_ref, o_ref):
    h = hp_ref[:, :]
    mean = (p0_ref[:, :] + p1_ref[:, :]) * inv_ref[:, :]
    z = _conv(h, mean, wl_ref, wr_ref, bl_ref, br_ref)
    z = z + h
    mu = jnp.mean(z, axis=1, keepdims=True)
    var = jnp.mean((z - mu) ** 2, axis=1, keepdims=True)
    z = (z - mu) * lax.rsqrt(var + 1e-5)
    z = jnp.maximum(z, 0.0)
    logits = jnp.dot(z, woutp_ref[:, :], preferred_element_type=jnp.float32)
    logits = logits + boutp_ref[:, :]
    o_ref[:, :] = logits[:, :2]


_ROW = lambda i: (i, 0)
_FIX = lambda i: (0, 0)


def _tc_layer(residual, h, p0, p1, inv_cnt, Wl, Wr, bl, br):
    return pl.pallas_call(
        functools.partial(_tc_layer_body, residual),
        grid=(N // BR,),
        in_specs=[
            pl.BlockSpec((BR, D), _ROW),
            pl.BlockSpec((BR, D), _ROW),
            pl.BlockSpec((BR, D), _ROW),
            pl.BlockSpec((BR, 1), _ROW),
            pl.BlockSpec((D, D), _FIX),
            pl.BlockSpec((D, D), _FIX),
            pl.BlockSpec((1, D), _FIX),
            pl.BlockSpec((1, D), _FIX),
        ],
        out_specs=pl.BlockSpec((BR, D), _ROW),
        out_shape=jax.ShapeDtypeStruct((N, D), jnp.float32),
    )(h, p0, p1, inv_cnt, Wl, Wr, bl.reshape(1, D), br.reshape(1, D))


def _tc_last(h, p0, p1, inv_cnt, Wl, Wr, bl, br, Woutp, boutp):
    return pl.pallas_call(
        _tc_last_body,
        grid=(N // BR,),
        in_specs=[
            pl.BlockSpec((BR, D), _ROW),
            pl.BlockSpec((BR, D), _ROW),
            pl.BlockSpec((BR, D), _ROW),
            pl.BlockSpec((BR, 1), _ROW),
            pl.BlockSpec((D, D), _FIX),
            pl.BlockSpec((D, D), _FIX),
            pl.BlockSpec((1, D), _FIX),
            pl.BlockSpec((1, D), _FIX),
            pl.BlockSpec((D, D), _FIX),
            pl.BlockSpec((1, D), _FIX),
        ],
        out_specs=pl.BlockSpec((BR, 2), _ROW),
        out_shape=jax.ShapeDtypeStruct((N, 2), jnp.float32),
    )(h, p0, p1, inv_cnt, Wl, Wr, bl.reshape(1, D), br.reshape(1, D),
      Woutp, boutp)


def kernel(x, edge_index, Wl0, bl0, Wr0, br0, Wl1, bl1, Wr1, br1,
           Wl2, bl2, Wr2, br2, Wout, bout):
    f32 = jnp.float32
    x = x.astype(f32)
    src = edge_index[0].astype(jnp.int32)
    dst = edge_index[1].astype(jnp.int32)
    srcp = jnp.concatenate(
        [src, jnp.zeros((EPAD - E,), jnp.int32)]).reshape(-1, 1, CHUNK)
    dummy = N + jnp.arange(EPAD - E, dtype=jnp.int32) % (NPAD - N)
    dstp = jnp.concatenate([dst, dummy]).reshape(-1, 1, CHUNK)
    zrows = jnp.zeros((RPT, D), f32)
    iota80 = jnp.arange(HR, dtype=jnp.int32)
    Woutp = jnp.zeros((D, D), f32).at[:, :2].set(Wout.astype(f32))
    boutp = jnp.zeros((1, D), f32).at[:, :2].set(bout.astype(f32)[None, :])

    c0, c1 = _sc_degree(dstp, zrows, iota80)
    cnt = (c0 + c1).reshape(NPAD)[:N].reshape(N, 1)
    inv_cnt = 1.0 / jnp.maximum(cnt, 1.0)

    h = x
    p0, p1 = _sc_segsum(h, srcp, dstp, zrows)
    h = _tc_layer(False, h, p0, p1, inv_cnt, Wl0, Wr0, bl0, br0)
    p0, p1 = _sc_segsum(h, srcp, dstp, zrows)
    h = _tc_layer(True, h, p0, p1, inv_cnt, Wl1, Wr1, bl1, br1)
    p0, p1 = _sc_segsum(h, srcp, dstp, zrows)
    return _tc_last(h, p0, p1, inv_cnt, Wl2, Wr2, bl2, br2, Woutp, boutp)


# speedup vs baseline: 2.2506x; 2.1542x over previous
import functools

import jax
import jax.numpy as jnp
from jax import lax
from jax.experimental import pallas as pl
from jax.experimental.pallas import tpu as pltpu
from jax.experimental.pallas import tpu_sc as plsc

N = 10000
E = 320000
D = 128
NC, NS = 2, 16
NT = NC * NS
CHUNK = 128
EPT = 10240
EPAD = NT * EPT
NITER = EPT // CHUNK
NPAD = 10240
RPT = NPAD // NS
OPT = 624
HR = NPAD // D
BR = 1000


NBUF = 2
NITER0 = 124
NITER1 = 36
NHALF0 = NITER0 // 2
NHALF1 = NITER1 // 2


def _run_edges(h, srcp3, dstp3, acc, sidx, didx, bufs, sems, base, niter):
    nhalf = niter // 2
    ngr = nhalf // NBUF
    for half in range(2):
        hbase = base + half * nhalf
        pltpu.sync_copy(srcp3.at[pl.ds(hbase, nhalf)],
                        sidx.at[pl.ds(0, nhalf)])
        pltpu.sync_copy(dstp3.at[pl.ds(hbase, nhalf)],
                        didx.at[pl.ds(0, nhalf)])

        for b in range(NBUF):
            pltpu.async_copy(h.at[sidx.at[b, 0]], bufs[b], sems[b])

        def group(g, carry):
            for b in range(NBUF):
                i = g * NBUF + b
                pltpu.make_async_copy(h.at[sidx.at[i, 0]], bufs[b],
                                      sems[b]).wait()
                pltpu.sync_copy(bufs[b], acc.at[didx.at[i, 0]], add=True)

                @pl.when(g < ngr - 1)
                def _():
                    pltpu.async_copy(h.at[sidx.at[i + NBUF, 0]], bufs[b],
                                     sems[b])
            return carry

        lax.fori_loop(0, ngr, group, 0)


def _sc_segsum_body(h, srcp3, dstp3, zrows, out0, out1,
                    acc, sidx, didx, r0, r1, sg0, sg1):
    bufs = [r0, r1]
    sems = [sg0, sg1]
    c = lax.axis_index("c")
    s = lax.axis_index("s")

    with jax.named_scope("zero"):
        pltpu.sync_copy(zrows, acc.at[pl.ds(s * RPT, RPT)])
        plsc.subcore_barrier()

    with jax.named_scope("edges"):
        @pl.when(c == 0)
        def _():
            _run_edges(h, srcp3, dstp3, acc, sidx, didx, bufs, sems,
                       s * NITER0, NITER0)

        @pl.when(c == 1)
        def _():
            _run_edges(h, srcp3, dstp3, acc, sidx, didx, bufs, sems,
                       NS * NITER0 + s * NITER1, NITER1)

    with jax.named_scope("waitall"):
        plsc.subcore_barrier()

    rows = pl.ds(s * OPT, OPT)
    tail = pl.ds(NS * OPT, N - NS * OPT)

    with jax.named_scope("writeout"):
        @pl.when(c == 0)
        def _():
            pltpu.sync_copy(acc.at[rows], out0.at[rows])

        @pl.when(c == 1)
        def _():
            pltpu.sync_copy(acc.at[rows], out1.at[rows])

        @pl.when((c == 0) & (s == NS - 1))
        def _():
            pltpu.sync_copy(acc.at[tail], out0.at[tail])

        @pl.when((c == 1) & (s == NS - 1))
        def _():
            pltpu.sync_copy(acc.at[tail], out1.at[tail])


_sc_segsum = pl.kernel(
    _sc_segsum_body,
    out_type=(jax.ShapeDtypeStruct((N, D), jnp.float32),
              jax.ShapeDtypeStruct((N, D), jnp.float32)),
    mesh=plsc.VectorSubcoreMesh(core_axis_name="c", subcore_axis_name="s"),
    scratch_types=(
        [pltpu.VMEM_SHARED((NPAD, D), jnp.float32),
         pltpu.VMEM((NHALF0, 1, CHUNK), jnp.int32),
         pltpu.VMEM((NHALF0, 1, CHUNK), jnp.int32)]
        + [pltpu.VMEM((CHUNK, D), jnp.float32)] * NBUF
        + [pltpu.SemaphoreType.DMA] * NBUF
    ),
)


def _sc_degree_body(dstp, zrows, iota_hbm, out0, out1,
                    acc, hist, didx_v, idx80, sem):
    c = lax.axis_index("c")
    s = lax.axis_index("s")
    tile = c * NS + s

    @pl.when(s < HR // 8)
    def _():
        pltpu.sync_copy(zrows.at[pl.ds(0, 8)], acc.at[pl.ds(s * 8, 8)])

    pltpu.sync_copy(zrows.at[pl.ds(0, HR)], hist)
    pltpu.sync_copy(iota_hbm, idx80)
    plsc.subcore_barrier()

    def step(i, carry):
        pltpu.sync_copy(dstp.at[tile * NITER + i, 0], didx_v)
        for k in range(CHUNK // 16):
            d16 = didx_v[pl.ds(k * 16, 16)]
            cnt, last = plsc.scan_count(d16)
            plsc.addupdate_scatter(
                hist,
                [lax.shift_right_logical(d16, 7),
                 lax.bitwise_and(d16, 127)],
                cnt.astype(jnp.float32),
                mask=last,
            )
        return carry

    lax.fori_loop(0, NITER, step, 0)
    pltpu.sync_copy(hist, acc.at[idx80], add=True)
    plsc.subcore_barrier()

    @pl.when((c == 0) & (s < HR // 8))
    def _():
        pltpu.sync_copy(acc.at[pl.ds(s * 8, 8)], out0.at[pl.ds(s * 8, 8)])

    @pl.when((c == 1) & (s < HR // 8))
    def _():
        pltpu.sync_copy(acc.at[pl.ds(s * 8, 8)], out1.at[pl.ds(s * 8, 8)])


_sc_degree = pl.kernel(
    _sc_degree_body,
    out_type=(jax.ShapeDtypeStruct((HR, D), jnp.float32),
              jax.ShapeDtypeStruct((HR, D), jnp.float32)),
    mesh=plsc.VectorSubcoreMesh(core_axis_name="c", subcore_axis_name="s"),
    scratch_types=[
        pltpu.VMEM_SHARED((HR, D), jnp.float32),
        pltpu.VMEM((HR, D), jnp.float32),
        pltpu.VMEM((CHUNK,), jnp.int32),
        pltpu.VMEM((HR,), jnp.int32),
        pltpu.SemaphoreType.DMA,
    ],
    compiler_params=pltpu.CompilerParams(needs_layout_passes=False),
)


def _conv(h, mean, wl_ref, wr_ref, bl_ref, br_ref):
    return (jnp.dot(mean, wl_ref[:, :], preferred_element_type=jnp.float32)
            + jnp.dot(h, wr_ref[:, :], preferred_element_type=jnp.float32)
            + bl_ref[:, :] + br_ref[:, :])


def _tc_layer_body(residual, hp_ref, p0_ref, p1_ref, inv_ref, wl_ref, wr_ref,
                   bl_ref, br_ref, o_ref):
    h = hp_ref[:, :]
    mean = (p0_ref[:, :] + p1_ref[:, :]) * inv_ref[:, :]
    z = _conv(h, mean, wl_ref, wr_ref, bl_ref, br_ref)
    if residual:
        z = z + h
        mu = jnp.mean(z, axis=1, keepdims=True)
        var = jnp.mean((z - mu) ** 2, axis=1, keepdims=True)
        z = (z - mu) * lax.rsqrt(var + 1e-5)
    o_ref[:, :] = jnp.maximum(z, 0.0)


def _tc_last_body(hp_ref, p0_ref, p1_ref, inv_ref, wl_ref, wr_ref,
                  bl_ref, br_ref, woutp_ref, boutp_ref, o_ref):
    h = hp_ref[:, :]
    mean = (p0_ref[:, :] + p1_ref[:, :]) * inv_ref[:, :]
    z = _conv(h, mean, wl_ref, wr_ref, bl_ref, br_ref)
    z = z + h
    mu = jnp.mean(z, axis=1, keepdims=True)
    var = jnp.mean((z - mu) ** 2, axis=1, keepdims=True)
    z = (z - mu) * lax.rsqrt(var + 1e-5)
    z = jnp.maximum(z, 0.0)
    logits = jnp.dot(z, woutp_ref[:, :], preferred_element_type=jnp.float32)
    logits = logits + boutp_ref[:, :]
    o_ref[:, :] = logits[:, :2]


_ROW = lambda i: (i, 0)
_FIX = lambda i: (0, 0)


def _tc_layer(residual, h, p0, p1, inv_cnt, Wl, Wr, bl, br):
    return pl.pallas_call(
        functools.partial(_tc_layer_body, residual),
        grid=(N // BR,),
        in_specs=[
            pl.BlockSpec((BR, D), _ROW),
            pl.BlockSpec((BR, D), _ROW),
            pl.BlockSpec((BR, D), _ROW),
            pl.BlockSpec((BR, 1), _ROW),
            pl.BlockSpec((D, D), _FIX),
            pl.BlockSpec((D, D), _FIX),
            pl.BlockSpec((1, D), _FIX),
            pl.BlockSpec((1, D), _FIX),
        ],
        out_specs=pl.BlockSpec((BR, D), _ROW),
        out_shape=jax.ShapeDtypeStruct((N, D), jnp.float32),
    )(h, p0, p1, inv_cnt, Wl, Wr, bl.reshape(1, D), br.reshape(1, D))


def _tc_last(h, p0, p1, inv_cnt, Wl, Wr, bl, br, Woutp, boutp):
    return pl.pallas_call(
        _tc_last_body,
        grid=(N // BR,),
        in_specs=[
            pl.BlockSpec((BR, D), _ROW),
            pl.BlockSpec((BR, D), _ROW),
            pl.BlockSpec((BR, D), _ROW),
            pl.BlockSpec((BR, 1), _ROW),
            pl.BlockSpec((D, D), _FIX),
            pl.BlockSpec((D, D), _FIX),
            pl.BlockSpec((1, D), _FIX),
            pl.BlockSpec((1, D), _FIX),
            pl.BlockSpec((D, D), _FIX),
            pl.BlockSpec((1, D), _FIX),
        ],
        out_specs=pl.BlockSpec((BR, 2), _ROW),
        out_shape=jax.ShapeDtypeStruct((N, 2), jnp.float32),
    )(h, p0, p1, inv_cnt, Wl, Wr, bl.reshape(1, D), br.reshape(1, D),
      Woutp, boutp)


def kernel(x, edge_index, Wl0, bl0, Wr0, br0, Wl1, bl1, Wr1, br1,
           Wl2, bl2, Wr2, br2, Wout, bout):
    f32 = jnp.float32
    x = x.astype(f32)
    src = edge_index[0].astype(jnp.int32)
    dst = edge_index[1].astype(jnp.int32)
    dummy_src = jnp.arange(EPAD - E, dtype=jnp.int32) % N
    srcp = jnp.concatenate([src, dummy_src]).reshape(-1, 1, CHUNK)
    dummy = N + jnp.arange(EPAD - E, dtype=jnp.int32) % (NPAD - N)
    dstp = jnp.concatenate([dst, dummy]).reshape(-1, 1, CHUNK)
    zrows = jnp.zeros((RPT, D), f32)
    iota80 = jnp.arange(HR, dtype=jnp.int32)
    Woutp = jnp.zeros((D, D), f32).at[:, :2].set(Wout.astype(f32))
    boutp = jnp.zeros((1, D), f32).at[:, :2].set(bout.astype(f32)[None, :])

    c0, c1 = _sc_degree(dstp, zrows, iota80)
    cnt = (c0 + c1).reshape(NPAD)[:N].reshape(N, 1)
    inv_cnt = 1.0 / jnp.maximum(cnt, 1.0)

    h = x
    p0, p1 = _sc_segsum(h, srcp, dstp, zrows)
    h = _tc_layer(False, h, p0, p1, inv_cnt, Wl0, Wr0, bl0, br0)
    p0, p1 = _sc_segsum(h, srcp, dstp, zrows)
    h = _tc_layer(True, h, p0, p1, inv_cnt, Wl1, Wr1, bl1, br1)
    p0, p1 = _sc_segsum(h, srcp, dstp, zrows)
    return _tc_last(h, p0, p1, inv_cnt, Wl2, Wr2, bl2, br2, Woutp, boutp)


# speedup vs baseline: 2.9375x; 1.3052x over previous
import functools

import jax
import jax.numpy as jnp
from jax import lax
from jax.experimental import pallas as pl
from jax.experimental.pallas import tpu as pltpu
from jax.experimental.pallas import tpu_sc as plsc

N = 10000
E = 320000
D = 128
NC, NS = 2, 16
NT = NC * NS
CHUNK = 128
EPT = 10240
EPAD = NT * EPT
NITER = EPT // CHUNK
NPAD = 10240
RPT = NPAD // NS
OPT = 624
HR = NPAD // D
BR = 1000


NBUF = 2
NITER0 = NITER
NITER1 = NITER
NHALF0 = NITER0 // 2


def _run_edges(h, srcp3, dstp3, acc, sidx, didx, bufs, sems, base, niter):
    nhalf = niter // 2
    ngr = nhalf // NBUF
    for half in range(2):
        hbase = base + half * nhalf
        pltpu.sync_copy(srcp3.at[pl.ds(hbase, nhalf)],
                        sidx.at[pl.ds(0, nhalf)])
        pltpu.sync_copy(dstp3.at[pl.ds(hbase, nhalf)],
                        didx.at[pl.ds(0, nhalf)])

        for b in range(NBUF):
            pltpu.async_copy(h.at[sidx.at[b, 0]], bufs[b], sems[b])

        def group(g, carry):
            for b in range(NBUF):
                i = g * NBUF + b
                pltpu.make_async_copy(h.at[sidx.at[i, 0]], bufs[b],
                                      sems[b]).wait()
                pltpu.sync_copy(bufs[b], acc.at[didx.at[i, 0]], add=True)

                @pl.when(g < ngr - 1)
                def _():
                    pltpu.async_copy(h.at[sidx.at[i + NBUF, 0]], bufs[b],
                                     sems[b])
            return carry

        lax.fori_loop(0, ngr, group, 0)


def _sc_segsum_body(h, srcp3, dstp3, zrows, out0, out1,
                    acc, sidx, didx, r0, r1, sg0, sg1):
    bufs = [r0, r1]
    sems = [sg0, sg1]
    c = lax.axis_index("c")
    s = lax.axis_index("s")

    with jax.named_scope("zero"):
        pltpu.sync_copy(zrows, acc.at[pl.ds(s * RPT, RPT)])
        plsc.subcore_barrier()

    with jax.named_scope("edges"):
        @pl.when(c == 0)
        def _():
            _run_edges(h, srcp3, dstp3, acc, sidx, didx, bufs, sems,
                       s * NITER0, NITER0)

        @pl.when(c == 1)
        def _():
            _run_edges(h, srcp3, dstp3, acc, sidx, didx, bufs, sems,
                       NS * NITER0 + s * NITER1, NITER1)

    with jax.named_scope("waitall"):
        plsc.subcore_barrier()

    rows = pl.ds(s * OPT, OPT)
    tail = pl.ds(NS * OPT, N - NS * OPT)

    with jax.named_scope("writeout"):
        @pl.when(c == 0)
        def _():
            pltpu.sync_copy(acc.at[rows], out0.at[rows])

        @pl.when(c == 1)
        def _():
            pltpu.sync_copy(acc.at[rows], out1.at[rows])

        @pl.when((c == 0) & (s == NS - 1))
        def _():
            pltpu.sync_copy(acc.at[tail], out0.at[tail])

        @pl.when((c == 1) & (s == NS - 1))
        def _():
            pltpu.sync_copy(acc.at[tail], out1.at[tail])


_sc_segsum = pl.kernel(
    _sc_segsum_body,
    out_type=(jax.ShapeDtypeStruct((N, D), jnp.float32),
              jax.ShapeDtypeStruct((N, D), jnp.float32)),
    mesh=plsc.VectorSubcoreMesh(core_axis_name="c", subcore_axis_name="s"),
    scratch_types=(
        [pltpu.VMEM_SHARED((NPAD, D), jnp.float32),
         pltpu.VMEM((NHALF0, 1, CHUNK), jnp.int32),
         pltpu.VMEM((NHALF0, 1, CHUNK), jnp.int32)]
        + [pltpu.VMEM((CHUNK, D), jnp.float32)] * NBUF
        + [pltpu.SemaphoreType.DMA] * NBUF
    ),
)


def _sc_degree_body(dstp, zrows, iota_hbm, out0, out1,
                    acc, hist, didx_v, idx80, sem):
    c = lax.axis_index("c")
    s = lax.axis_index("s")
    tile = c * NS + s

    @pl.when(s < HR // 8)
    def _():
        pltpu.sync_copy(zrows.at[pl.ds(0, 8)], acc.at[pl.ds(s * 8, 8)])

    pltpu.sync_copy(zrows.at[pl.ds(0, HR)], hist)
    pltpu.sync_copy(iota_hbm, idx80)
    plsc.subcore_barrier()

    def step(i, carry):
        pltpu.sync_copy(dstp.at[tile * NITER + i, 0], didx_v)
        for k in range(CHUNK // 16):
            d16 = didx_v[pl.ds(k * 16, 16)]
            cnt, last = plsc.scan_count(d16)
            plsc.addupdate_scatter(
                hist,
                [lax.shift_right_logical(d16, 7),
                 lax.bitwise_and(d16, 127)],
                cnt.astype(jnp.float32),
                mask=last,
            )
        return carry

    lax.fori_loop(0, NITER, step, 0)
    pltpu.sync_copy(hist, acc.at[idx80], add=True)
    plsc.subcore_barrier()

    @pl.when((c == 0) & (s < HR // 8))
    def _():
        pltpu.sync_copy(acc.at[pl.ds(s * 8, 8)], out0.at[pl.ds(s * 8, 8)])

    @pl.when((c == 1) & (s < HR // 8))
    def _():
        pltpu.sync_copy(acc.at[pl.ds(s * 8, 8)], out1.at[pl.ds(s * 8, 8)])


_sc_degree = pl.kernel(
    _sc_degree_body,
    out_type=(jax.ShapeDtypeStruct((HR, D), jnp.float32),
              jax.ShapeDtypeStruct((HR, D), jnp.float32)),
    mesh=plsc.VectorSubcoreMesh(core_axis_name="c", subcore_axis_name="s"),
    scratch_types=[
        pltpu.VMEM_SHARED((HR, D), jnp.float32),
        pltpu.VMEM((HR, D), jnp.float32),
        pltpu.VMEM((CHUNK,), jnp.int32),
        pltpu.VMEM((HR,), jnp.int32),
        pltpu.SemaphoreType.DMA,
    ],
    compiler_params=pltpu.CompilerParams(needs_layout_passes=False),
)


def _conv(h, mean, wl_ref, wr_ref, bl_ref, br_ref):
    return (jnp.dot(mean, wl_ref[:, :], preferred_element_type=jnp.float32)
            + jnp.dot(h, wr_ref[:, :], preferred_element_type=jnp.float32)
            + bl_ref[:, :] + br_ref[:, :])


def _tc_layer_body(residual, hp_ref, p0_ref, p1_ref, inv_ref, wl_ref, wr_ref,
                   bl_ref, br_ref, o_ref):
    h = hp_ref[:, :]
    mean = (p0_ref[:, :] + p1_ref[:, :]) * inv_ref[:, :]
    z = _conv(h, mean, wl_ref, wr_ref, bl_ref, br_ref)
    if residual:
        z = z + h
        mu = jnp.mean(z, axis=1, keepdims=True)
        var = jnp.mean((z - mu) ** 2, axis=1, keepdims=True)
        z = (z - mu) * lax.rsqrt(var + 1e-5)
    o_ref[:, :] = jnp.maximum(z, 0.0)


def _tc_last_body(hp_ref, p0_ref, p1_ref, inv_ref, wl_ref, wr_ref,
                  bl_ref, br_ref, woutp_ref, boutp_ref, o_ref):
    h = hp_ref[:, :]
    mean = (p0_ref[:, :] + p1_ref[:, :]) * inv_ref[:, :]
    z = _conv(h, mean, wl_ref, wr_ref, bl_ref, br_ref)
    z = z + h
    mu = jnp.mean(z, axis=1, keepdims=True)
    var = jnp.mean((z - mu) ** 2, axis=1, keepdims=True)
    z = (z - mu) * lax.rsqrt(var + 1e-5)
    z = jnp.maximum(z, 0.0)
    logits = jnp.dot(z, woutp_ref[:, :], preferred_element_type=jnp.float32)
    logits = logits + boutp_ref[:, :]
    o_ref[:, :] = logits[:, :2]


_ROW = lambda i: (i, 0)
_FIX = lambda i: (0, 0)


def _tc_layer(residual, h, p0, p1, inv_cnt, Wl, Wr, bl, br):
    return pl.pallas_call(
        functools.partial(_tc_layer_body, residual),
        grid=(N // BR,),
        in_specs=[
            pl.BlockSpec((BR, D), _ROW),
            pl.BlockSpec((BR, D), _ROW),
            pl.BlockSpec((BR, D), _ROW),
            pl.BlockSpec((BR, 1), _ROW),
            pl.BlockSpec((D, D), _FIX),
            pl.BlockSpec((D, D), _FIX),
            pl.BlockSpec((1, D), _FIX),
            pl.BlockSpec((1, D), _FIX),
        ],
        out_specs=pl.BlockSpec((BR, D), _ROW),
        out_shape=jax.ShapeDtypeStruct((N, D), jnp.float32),
    )(h, p0, p1, inv_cnt, Wl, Wr, bl.reshape(1, D), br.reshape(1, D))


def _tc_last(h, p0, p1, inv_cnt, Wl, Wr, bl, br, Woutp, boutp):
    return pl.pallas_call(
        _tc_last_body,
        grid=(N // BR,),
        in_specs=[
            pl.BlockSpec((BR, D), _ROW),
            pl.BlockSpec((BR, D), _ROW),
            pl.BlockSpec((BR, D), _ROW),
            pl.BlockSpec((BR, 1), _ROW),
            pl.BlockSpec((D, D), _FIX),
            pl.BlockSpec((D, D), _FIX),
            pl.BlockSpec((1, D), _FIX),
            pl.BlockSpec((1, D), _FIX),
            pl.BlockSpec((D, D), _FIX),
            pl.BlockSpec((1, D), _FIX),
        ],
        out_specs=pl.BlockSpec((BR, 2), _ROW),
        out_shape=jax.ShapeDtypeStruct((N, 2), jnp.float32),
    )(h, p0, p1, inv_cnt, Wl, Wr, bl.reshape(1, D), br.reshape(1, D),
      Woutp, boutp)


def kernel(x, edge_index, Wl0, bl0, Wr0, br0, Wl1, bl1, Wr1, br1,
           Wl2, bl2, Wr2, br2, Wout, bout):
    f32 = jnp.float32
    x = x.astype(f32)
    src = edge_index[0].astype(jnp.int32)
    dst = edge_index[1].astype(jnp.int32)
    dummy_src = jnp.arange(EPAD - E, dtype=jnp.int32) % N
    srcp = jnp.concatenate([src, dummy_src]).reshape(-1, 1, CHUNK)
    dummy = N + jnp.arange(EPAD - E, dtype=jnp.int32) % (NPAD - N)
    dstp = jnp.concatenate([dst, dummy]).reshape(-1, 1, CHUNK)
    zrows = jnp.zeros((RPT, D), f32)
    iota80 = jnp.arange(HR, dtype=jnp.int32)
    Woutp = jnp.zeros((D, D), f32).at[:, :2].set(Wout.astype(f32))
    boutp = jnp.zeros((1, D), f32).at[:, :2].set(bout.astype(f32)[None, :])

    c0, c1 = _sc_degree(dstp, zrows, iota80)
    cnt = (c0 + c1).reshape(NPAD)[:N].reshape(N, 1)
    inv_cnt = 1.0 / jnp.maximum(cnt, 1.0)

    h = x
    p0, p1 = _sc_segsum(h, srcp, dstp, zrows)
    h = _tc_layer(False, h, p0, p1, inv_cnt, Wl0, Wr0, bl0, br0)
    p0, p1 = _sc_segsum(h, srcp, dstp, zrows)
    h = _tc_layer(True, h, p0, p1, inv_cnt, Wl1, Wr1, bl1, br1)
    p0, p1 = _sc_segsum(h, srcp, dstp, zrows)
    return _tc_last(h, p0, p1, inv_cnt, Wl2, Wr2, bl2, br2, Woutp, boutp)


# speedup vs baseline: 3.1960x; 1.0880x over previous
import functools

import jax
import jax.numpy as jnp
from jax import lax
from jax.experimental import pallas as pl
from jax.experimental.pallas import tpu as pltpu
from jax.experimental.pallas import tpu_sc as plsc

N = 10000
E = 320000
D = 128
NC, NS = 2, 16
NT = NC * NS
CHUNK = 128
EPT = 10240
EPAD = NT * EPT
NITER = EPT // CHUNK
NPAD = 10240
RPT = NPAD // NS
OPT = 624
HR = NPAD // D
BR = 1000


NBUF = 2
NITER0 = NITER
NITER1 = NITER
NHALF0 = NITER0 // 2


def _run_edges(h, srcp3, dstp3, acc, sidx, didx, bufs, sems, base, niter):
    nhalf = niter // 2
    ngr = nhalf // NBUF
    for half in range(2):
        hbase = base + half * nhalf
        pltpu.sync_copy(srcp3.at[pl.ds(hbase, nhalf)],
                        sidx.at[pl.ds(0, nhalf)])
        pltpu.sync_copy(dstp3.at[pl.ds(hbase, nhalf)],
                        didx.at[pl.ds(0, nhalf)])

        for b in range(NBUF):
            pltpu.async_copy(h.at[sidx.at[b, 0]], bufs[b], sems[b])

        def group(g, carry):
            for b in range(NBUF):
                i = g * NBUF + b
                pltpu.make_async_copy(h.at[sidx.at[i, 0]], bufs[b],
                                      sems[b]).wait()
                pltpu.sync_copy(bufs[b], acc.at[didx.at[i, 0]], add=True)

                @pl.when(g < ngr - 1)
                def _():
                    pltpu.async_copy(h.at[sidx.at[i + NBUF, 0]], bufs[b],
                                     sems[b])
            return carry

        lax.fori_loop(0, ngr, group, 0)


def _sc_segsum_body(h, srcp3, dstp3, zrows, out0, out1,
                    acc, sidx, didx, r0, r1, sg0, sg1):
    bufs = [r0, r1]
    sems = [sg0, sg1]
    c = lax.axis_index("c")
    s = lax.axis_index("s")

    with jax.named_scope("zero"):
        pltpu.sync_copy(zrows, acc.at[pl.ds(s * RPT, RPT)])
        plsc.subcore_barrier()

    with jax.named_scope("edges"):
        @pl.when(c == 0)
        def _():
            _run_edges(h, srcp3, dstp3, acc, sidx, didx, bufs, sems,
                       s * NITER0, NITER0)

        @pl.when(c == 1)
        def _():
            _run_edges(h, srcp3, dstp3, acc, sidx, didx, bufs, sems,
                       NS * NITER0 + s * NITER1, NITER1)

    with jax.named_scope("waitall"):
        plsc.subcore_barrier()

    rows = pl.ds(s * OPT, OPT)
    tail = pl.ds(NS * OPT, N - NS * OPT)

    with jax.named_scope("writeout"):
        @pl.when(c == 0)
        def _():
            pltpu.sync_copy(acc.at[rows], out0.at[rows])

        @pl.when(c == 1)
        def _():
            pltpu.sync_copy(acc.at[rows], out1.at[rows])

        @pl.when((c == 0) & (s == NS - 1))
        def _():
            pltpu.sync_copy(acc.at[tail], out0.at[tail])

        @pl.when((c == 1) & (s == NS - 1))
        def _():
            pltpu.sync_copy(acc.at[tail], out1.at[tail])


_sc_segsum = pl.kernel(
    _sc_segsum_body,
    out_type=(jax.ShapeDtypeStruct((N, D), jnp.float32),
              jax.ShapeDtypeStruct((N, D), jnp.float32)),
    mesh=plsc.VectorSubcoreMesh(core_axis_name="c", subcore_axis_name="s"),
    scratch_types=(
        [pltpu.VMEM_SHARED((NPAD, D), jnp.float32),
         pltpu.VMEM((NHALF0, 1, CHUNK), jnp.int32),
         pltpu.VMEM((NHALF0, 1, CHUNK), jnp.int32)]
        + [pltpu.VMEM((CHUNK, D), jnp.float32)] * NBUF
        + [pltpu.SemaphoreType.DMA] * NBUF
    ),
)


def _sc_degree_body(dstp, iota_hbm, out0, out1,
                    acc, hist, didx, idx80, sem):
    c = lax.axis_index("c")
    s = lax.axis_index("s")
    tile = c * NS + s

    cp = pltpu.async_copy(dstp.at[pl.ds(tile * NITER, NITER)], didx, sem)
    pltpu.sync_copy(iota_hbm, idx80)
    z16 = jnp.zeros((16,), jnp.float32)
    for r in range(HR):
        for k in range(D // 16):
            hist[r, pl.ds(k * 16, 16)] = z16

    @pl.when(s < HR // 8)
    def _():
        pltpu.sync_copy(hist.at[pl.ds(0, 8)], acc.at[pl.ds(s * 8, 8)])

    cp.wait()
    plsc.subcore_barrier()

    def step(i, carry):
        for k in range(CHUNK // 16):
            d16 = didx[i, 0, pl.ds(k * 16, 16)]
            cnt, last = plsc.scan_count(d16)
            plsc.addupdate_scatter(
                hist,
                [lax.shift_right_logical(d16, 7),
                 lax.bitwise_and(d16, 127)],
                cnt.astype(jnp.float32),
                mask=last,
            )
        return carry

    lax.fori_loop(0, NITER, step, 0)
    pltpu.sync_copy(hist, acc.at[idx80], add=True)
    plsc.subcore_barrier()

    @pl.when((c == 0) & (s < HR // 8))
    def _():
        pltpu.sync_copy(acc.at[pl.ds(s * 8, 8)], out0.at[pl.ds(s * 8, 8)])

    @pl.when((c == 1) & (s < HR // 8))
    def _():
        pltpu.sync_copy(acc.at[pl.ds(s * 8, 8)], out1.at[pl.ds(s * 8, 8)])


_sc_degree = pl.kernel(
    _sc_degree_body,
    out_type=(jax.ShapeDtypeStruct((HR, D), jnp.float32),
              jax.ShapeDtypeStruct((HR, D), jnp.float32)),
    mesh=plsc.VectorSubcoreMesh(core_axis_name="c", subcore_axis_name="s"),
    scratch_types=[
        pltpu.VMEM_SHARED((HR, D), jnp.float32),
        pltpu.VMEM((HR, D), jnp.float32),
        pltpu.VMEM((NITER, 1, CHUNK), jnp.int32),
        pltpu.VMEM((HR,), jnp.int32),
        pltpu.SemaphoreType.DMA,
    ],
    compiler_params=pltpu.CompilerParams(needs_layout_passes=False),
)


def _conv(h, mean, wl_ref, wr_ref, bl_ref, br_ref):
    return (jnp.dot(mean, wl_ref[:, :], preferred_element_type=jnp.float32)
            + jnp.dot(h, wr_ref[:, :], preferred_element_type=jnp.float32)
            + bl_ref[:, :] + br_ref[:, :])


def _tc_layer_body(residual, hp_ref, p0_ref, p1_ref, inv_ref, wl_ref, wr_ref,
                   bl_ref, br_ref, o_ref):
    h = hp_ref[:, :]
    mean = (p0_ref[:, :] + p1_ref[:, :]) * inv_ref[:, :]
    z = _conv(h, mean, wl_ref, wr_ref, bl_ref, br_ref)
    if residual:
        z = z + h
        mu = jnp.mean(z, axis=1, keepdims=True)
        var = jnp.mean((z - mu) ** 2, axis=1, keepdims=True)
        z = (z - mu) * lax.rsqrt(var + 1e-5)
    o_ref[:, :] = jnp.maximum(z, 0.0)


def _tc_last_body(hp_ref, p0_ref, p1_ref, inv_ref, wl_ref, wr_ref,
                  bl_ref, br_ref, woutp_ref, boutp_ref, o_ref):
    h = hp_ref[:, :]
    mean = (p0_ref[:, :] + p1_ref[:, :]) * inv_ref[:, :]
    z = _conv(h, mean, wl_ref, wr_ref, bl_ref, br_ref)
    z = z + h
    mu = jnp.mean(z, axis=1, keepdims=True)
    var = jnp.mean((z - mu) ** 2, axis=1, keepdims=True)
    z = (z - mu) * lax.rsqrt(var + 1e-5)
    z = jnp.maximum(z, 0.0)
    logits = jnp.dot(z, woutp_ref[:, :], preferred_element_type=jnp.float32)
    logits = logits + boutp_ref[:, :]
    o_ref[:, :] = logits[:, :2]


_ROW = lambda i: (i, 0)
_FIX = lambda i: (0, 0)


def _tc_layer(residual, h, p0, p1, inv_cnt, Wl, Wr, bl, br):
    return pl.pallas_call(
        functools.partial(_tc_layer_body, residual),
        grid=(N // BR,),
        in_specs=[
            pl.BlockSpec((BR, D), _ROW),
            pl.BlockSpec((BR, D), _ROW),
            pl.BlockSpec((BR, D), _ROW),
            pl.BlockSpec((BR, 1), _ROW),
            pl.BlockSpec((D, D), _FIX),
            pl.BlockSpec((D, D), _FIX),
            pl.BlockSpec((1, D), _FIX),
            pl.BlockSpec((1, D), _FIX),
        ],
        out_specs=pl.BlockSpec((BR, D), _ROW),
        out_shape=jax.ShapeDtypeStruct((N, D), jnp.float32),
    )(h, p0, p1, inv_cnt, Wl, Wr, bl.reshape(1, D), br.reshape(1, D))


def _tc_last(h, p0, p1, inv_cnt, Wl, Wr, bl, br, Woutp, boutp):
    return pl.pallas_call(
        _tc_last_body,
        grid=(N // BR,),
        in_specs=[
            pl.BlockSpec((BR, D), _ROW),
            pl.BlockSpec((BR, D), _ROW),
            pl.BlockSpec((BR, D), _ROW),
            pl.BlockSpec((BR, 1), _ROW),
            pl.BlockSpec((D, D), _FIX),
            pl.BlockSpec((D, D), _FIX),
            pl.BlockSpec((1, D), _FIX),
            pl.BlockSpec((1, D), _FIX),
            pl.BlockSpec((D, D), _FIX),
            pl.BlockSpec((1, D), _FIX),
        ],
        out_specs=pl.BlockSpec((BR, 2), _ROW),
        out_shape=jax.ShapeDtypeStruct((N, 2), jnp.float32),
    )(h, p0, p1, inv_cnt, Wl, Wr, bl.reshape(1, D), br.reshape(1, D),
      Woutp, boutp)


def kernel(x, edge_index, Wl0, bl0, Wr0, br0, Wl1, bl1, Wr1, br1,
           Wl2, bl2, Wr2, br2, Wout, bout):
    f32 = jnp.float32
    x = x.astype(f32)
    src = edge_index[0].astype(jnp.int32)
    dst = edge_index[1].astype(jnp.int32)
    dummy_src = jnp.arange(EPAD - E, dtype=jnp.int32) % N
    srcp = jnp.concatenate([src, dummy_src]).reshape(-1, 1, CHUNK)
    dummy = N + jnp.arange(EPAD - E, dtype=jnp.int32) % (NPAD - N)
    dstp = jnp.concatenate([dst, dummy]).reshape(-1, 1, CHUNK)
    zrows = jnp.zeros((RPT, D), f32)
    iota80 = jnp.arange(HR, dtype=jnp.int32)
    Woutp = jnp.zeros((D, D), f32).at[:, :2].set(Wout.astype(f32))
    boutp = jnp.zeros((1, D), f32).at[:, :2].set(bout.astype(f32)[None, :])

    c0, c1 = _sc_degree(dstp, iota80)
    cnt = (c0 + c1).reshape(NPAD)[:N].reshape(N, 1)
    inv_cnt = 1.0 / jnp.maximum(cnt, 1.0)

    h = x
    p0, p1 = _sc_segsum(h, srcp, dstp, zrows)
    h = _tc_layer(False, h, p0, p1, inv_cnt, Wl0, Wr0, bl0, br0)
    p0, p1 = _sc_segsum(h, srcp, dstp, zrows)
    h = _tc_layer(True, h, p0, p1, inv_cnt, Wl1, Wr1, bl1, br1)
    p0, p1 = _sc_segsum(h, srcp, dstp, zrows)
    return _tc_last(h, p0, p1, inv_cnt, Wl2, Wr2, bl2, br2, Woutp, boutp)


# speedup vs baseline: 3.3011x; 1.0329x over previous
import functools

import jax
import jax.numpy as jnp
from jax import lax
from jax.experimental import pallas as pl
from jax.experimental.pallas import tpu as pltpu
from jax.experimental.pallas import tpu_sc as plsc

N = 10000
E = 320000
D = 128
NC, NS = 2, 16
NT = NC * NS
CHUNK = 120
NITER = 84
EPT = NITER * CHUNK
EPAD = NT * EPT
NPAD = 10240
RPT = NPAD // NS
OPT = 624
HR = NPAD // D
BR = 1000


NBUF = 3
NIB = 6


def _edge_ops(h, srcp3, dstp3, acc, rows, sidxb, didxb, sg, ss, si, base):
    def idx_issue(i, q):
        pltpu.async_copy(srcp3.at[base + i, 0], sidxb[q], si[q])
        pltpu.async_copy(dstp3.at[base + i, 0], didxb[q], si[q])

    def idx_wait(i, q):
        pltpu.make_async_copy(srcp3.at[base + i, 0], sidxb[q], si[q]).wait()
        pltpu.make_async_copy(dstp3.at[base + i, 0], didxb[q], si[q]).wait()

    def gather_issue(b, q):
        pltpu.async_copy(h.at[sidxb[q]], rows[b], sg[b])

    def gather_wait(b, q):
        pltpu.make_async_copy(h.at[sidxb[q]], rows[b], sg[b]).wait()

    def scatter_issue(b, q):
        pltpu.async_copy(rows[b], acc.at[didxb[q]], ss[b], add=True)

    def scatter_wait(b, q):
        pltpu.make_async_copy(rows[b], acc.at[didxb[q]], ss[b]).wait()

    return idx_issue, idx_wait, gather_issue, gather_wait, \
        scatter_issue, scatter_wait


def _run_edges(h, srcp3, dstp3, acc, rows, sidxb, didxb, sg, ss, si, base):
    idx_issue, idx_wait, gather_issue, gather_wait, scatter_issue, \
        scatter_wait = _edge_ops(h, srcp3, dstp3, acc, rows, sidxb, didxb,
                                 sg, ss, si, base)

    def slot(i, j, has_prev=True, has_g2=True, has_i4=True):
        b, q = j % NBUF, j % NIB
        gather_wait(b, q)
        scatter_issue(b, q)
        if has_prev:
            scatter_wait((j + 2) % NBUF, (j - 1) % NIB)
        if has_g2:
            idx_wait(i + 2, (j + 2) % NIB)
            gather_issue((j + 2) % NBUF, (j + 2) % NIB)
        if has_i4:
            idx_issue(i + 4, (j + 4) % NIB)

    for i in range(4):
        idx_issue(i, i)
    for i in range(2):
        idx_wait(i, i)
        gather_issue(i, i)

    for i in range(6):
        slot(i, i, has_prev=(i >= 1))

    def steady(g, carry):
        for j in range(6):
            slot(g * 6 + j, j)
        return carry

    lax.fori_loop(1, NITER // 6 - 1, steady, 0)

    for i in range(NITER - 6, NITER):
        slot(i, i % 6, has_g2=(i + 2 < NITER), has_i4=(i + 4 < NITER))
    scatter_wait((NITER - 1) % NBUF, (NITER - 1) % NIB)


def _sc_segsum_body(h, srcp3, dstp3, zrows, out0, out1, acc, *scratch):
    rows = list(scratch[0:3])
    sidxb = list(scratch[3:9])
    didxb = list(scratch[9:15])
    sg = list(scratch[15:18])
    ss = list(scratch[18:21])
    si = list(scratch[21:27])
    c = lax.axis_index("c")
    s = lax.axis_index("s")
    tile = c * NS + s

    with jax.named_scope("zero"):
        pltpu.sync_copy(zrows, acc.at[pl.ds(s * RPT, RPT)])
        plsc.subcore_barrier()

    with jax.named_scope("edges"):
        _run_edges(h, srcp3, dstp3, acc, rows, sidxb, didxb, sg, ss, si,
                   tile * NITER)

    with jax.named_scope("waitall"):
        plsc.subcore_barrier()

    rows = pl.ds(s * OPT, OPT)
    tail = pl.ds(NS * OPT, N - NS * OPT)

    with jax.named_scope("writeout"):
        @pl.when(c == 0)
        def _():
            pltpu.sync_copy(acc.at[rows], out0.at[rows])

        @pl.when(c == 1)
        def _():
            pltpu.sync_copy(acc.at[rows], out1.at[rows])

        @pl.when((c == 0) & (s == NS - 1))
        def _():
            pltpu.sync_copy(acc.at[tail], out0.at[tail])

        @pl.when((c == 1) & (s == NS - 1))
        def _():
            pltpu.sync_copy(acc.at[tail], out1.at[tail])


_sc_segsum = pl.kernel(
    _sc_segsum_body,
    out_type=(jax.ShapeDtypeStruct((N, D), jnp.float32),
              jax.ShapeDtypeStruct((N, D), jnp.float32)),
    mesh=plsc.VectorSubcoreMesh(core_axis_name="c", subcore_axis_name="s"),
    scratch_types=(
        [pltpu.VMEM_SHARED((NPAD, D), jnp.float32)]
        + [pltpu.VMEM((CHUNK, D), jnp.float32)] * NBUF
        + [pltpu.VMEM((CHUNK,), jnp.int32)] * (2 * NIB)
        + [pltpu.SemaphoreType.DMA] * (2 * NBUF + NIB)
    ),
)


def _sc_degree_body(dstf, iota_hbm, out0, out1,
                    acc, hist, didx, idx80, sem):
    c = lax.axis_index("c")
    s = lax.axis_index("s")
    tile = c * NS + s

    cp = pltpu.async_copy(dstf.at[pl.ds(tile * EPT, EPT)], didx, sem)
    pltpu.sync_copy(iota_hbm, idx80)
    z16 = jnp.zeros((16,), jnp.float32)
    for r in range(HR):
        for k in range(D // 16):
            hist[r, pl.ds(k * 16, 16)] = z16

    @pl.when(s < HR // 8)
    def _():
        pltpu.sync_copy(hist.at[pl.ds(0, 8)], acc.at[pl.ds(s * 8, 8)])

    cp.wait()
    plsc.subcore_barrier()

    def step(g, carry):
        for k in range(6):
            d16 = didx[pl.ds(g * 96 + k * 16, 16)]
            cnt, last = plsc.scan_count(d16)
            plsc.addupdate_scatter(
                hist,
                [lax.shift_right_logical(d16, 7),
                 lax.bitwise_and(d16, 127)],
                cnt.astype(jnp.float32),
                mask=last,
            )
        return carry

    lax.fori_loop(0, EPT // 96, step, 0)
    pltpu.sync_copy(hist, acc.at[idx80], add=True)
    plsc.subcore_barrier()

    @pl.when((c == 0) & (s < HR // 8))
    def _():
        pltpu.sync_copy(acc.at[pl.ds(s * 8, 8)], out0.at[pl.ds(s * 8, 8)])

    @pl.when((c == 1) & (s < HR // 8))
    def _():
        pltpu.sync_copy(acc.at[pl.ds(s * 8, 8)], out1.at[pl.ds(s * 8, 8)])


_sc_degree = pl.kernel(
    _sc_degree_body,
    out_type=(jax.ShapeDtypeStruct((HR, D), jnp.float32),
              jax.ShapeDtypeStruct((HR, D), jnp.float32)),
    mesh=plsc.VectorSubcoreMesh(core_axis_name="c", subcore_axis_name="s"),
    scratch_types=[
        pltpu.VMEM_SHARED((HR, D), jnp.float32),
        pltpu.VMEM((HR, D), jnp.float32),
        pltpu.VMEM((EPT,), jnp.int32),
        pltpu.VMEM((HR,), jnp.int32),
        pltpu.SemaphoreType.DMA,
    ],
    compiler_params=pltpu.CompilerParams(needs_layout_passes=False),
)


def _conv(h, mean, wl_ref, wr_ref, bl_ref, br_ref):
    return (jnp.dot(mean, wl_ref[:, :], preferred_element_type=jnp.float32)
            + jnp.dot(h, wr_ref[:, :], preferred_element_type=jnp.float32)
            + bl_ref[:, :] + br_ref[:, :])


def _tc_layer_body(residual, hp_ref, p0_ref, p1_ref, inv_ref, wl_ref, wr_ref,
                   bl_ref, br_ref, o_ref):
    h = hp_ref[:, :]
    mean = (p0_ref[:, :] + p1_ref[:, :]) * inv_ref[:, :]
    z = _conv(h, mean, wl_ref, wr_ref, bl_ref, br_ref)
    if residual:
        z = z + h
        mu = jnp.mean(z, axis=1, keepdims=True)
        var = jnp.mean((z - mu) ** 2, axis=1, keepdims=True)
        z = (z - mu) * lax.rsqrt(var + 1e-5)
    o_ref[:, :] = jnp.maximum(z, 0.0)


def _tc_last_body(hp_ref, p0_ref, p1_ref, inv_ref, wl_ref, wr_ref,
                  bl_ref, br_ref, woutp_ref, boutp_ref, o_ref):
    h = hp_ref[:, :]
    mean = (p0_ref[:, :] + p1_ref[:, :]) * inv_ref[:, :]
    z = _conv(h, mean, wl_ref, wr_ref, bl_ref, br_ref)
    z = z + h
    mu = jnp.mean(z, axis=1, keepdims=True)
    var = jnp.mean((z - mu) ** 2, axis=1, keepdims=True)
    z = (z - mu) * lax.rsqrt(var + 1e-5)
    z = jnp.maximum(z, 0.0)
    logits = jnp.dot(z, woutp_ref[:, :], preferred_element_type=jnp.float32)
    logits = logits + boutp_ref[:, :]
    o_ref[:, :] = logits[:, :2]


_ROW = lambda i: (i, 0)
_FIX = lambda i: (0, 0)


def _tc_layer(residual, h, p0, p1, inv_cnt, Wl, Wr, bl, br):
    return pl.pallas_call(
        functools.partial(_tc_layer_body, residual),
        grid=(N // BR,),
        in_specs=[
            pl.BlockSpec((BR, D), _ROW),
            pl.BlockSpec((BR, D), _ROW),
            pl.BlockSpec((BR, D), _ROW),
            pl.BlockSpec((BR, 1), _ROW),
            pl.BlockSpec((D, D), _FIX),
            pl.BlockSpec((D, D), _FIX),
            pl.BlockSpec((1, D), _FIX),
            pl.BlockSpec((1, D), _FIX),
        ],
        out_specs=pl.BlockSpec((BR, D), _ROW),
        out_shape=jax.ShapeDtypeStruct((N, D), jnp.float32),
    )(h, p0, p1, inv_cnt, Wl, Wr, bl.reshape(1, D), br.reshape(1, D))


def _tc_last(h, p0, p1, inv_cnt, Wl, Wr, bl, br, Woutp, boutp):
    return pl.pallas_call(
        _tc_last_body,
        grid=(N // BR,),
        in_specs=[
            pl.BlockSpec((BR, D), _ROW),
            pl.BlockSpec((BR, D), _ROW),
            pl.BlockSpec((BR, D), _ROW),
            pl.BlockSpec((BR, 1), _ROW),
            pl.BlockSpec((D, D), _FIX),
            pl.BlockSpec((D, D), _FIX),
            pl.BlockSpec((1, D), _FIX),
            pl.BlockSpec((1, D), _FIX),
            pl.BlockSpec((D, D), _FIX),
            pl.BlockSpec((1, D), _FIX),
        ],
        out_specs=pl.BlockSpec((BR, 2), _ROW),
        out_shape=jax.ShapeDtypeStruct((N, 2), jnp.float32),
    )(h, p0, p1, inv_cnt, Wl, Wr, bl.reshape(1, D), br.reshape(1, D),
      Woutp, boutp)


def kernel(x, edge_index, Wl0, bl0, Wr0, br0, Wl1, bl1, Wr1, br1,
           Wl2, bl2, Wr2, br2, Wout, bout):
    f32 = jnp.float32
    x = x.astype(f32)
    src = edge_index[0].astype(jnp.int32)
    dst = edge_index[1].astype(jnp.int32)
    dummy_src = jnp.arange(EPAD - E, dtype=jnp.int32) % N
    srcp = jnp.concatenate([src, dummy_src]).reshape(-1, 1, CHUNK)
    dummy = N + jnp.arange(EPAD - E, dtype=jnp.int32) % (NPAD - N)
    dstf = jnp.concatenate([dst, dummy])
    dstp = dstf.reshape(-1, 1, CHUNK)
    zrows = jnp.zeros((RPT, D), f32)
    iota80 = jnp.arange(HR, dtype=jnp.int32)
    Woutp = jnp.zeros((D, D), f32).at[:, :2].set(Wout.astype(f32))
    boutp = jnp.zeros((1, D), f32).at[:, :2].set(bout.astype(f32)[None, :])

    c0, c1 = _sc_degree(dstf, iota80)
    cnt = (c0 + c1).reshape(NPAD)[:N].reshape(N, 1)
    inv_cnt = 1.0 / jnp.maximum(cnt, 1.0)

    h = x
    p0, p1 = _sc_segsum(h, srcp, dstp, zrows)
    h = _tc_layer(False, h, p0, p1, inv_cnt, Wl0, Wr0, bl0, br0)
    p0, p1 = _sc_segsum(h, srcp, dstp, zrows)
    h = _tc_layer(True, h, p0, p1, inv_cnt, Wl1, Wr1, bl1, br1)
    p0, p1 = _sc_segsum(h, srcp, dstp, zrows)
    return _tc_last(h, p0, p1, inv_cnt, Wl2, Wr2, bl2, br2, Woutp, boutp)


# speedup vs baseline: 3.5208x; 1.0666x over previous
import functools

import jax
import jax.numpy as jnp
from jax import lax
from jax.experimental import pallas as pl
from jax.experimental.pallas import tpu as pltpu
from jax.experimental.pallas import tpu_sc as plsc

N = 10000
E = 320000
D = 128
NC, NS = 2, 16
NT = NC * NS
CHUNK = 120
NITER = 84
EPT = NITER * CHUNK
EPAD = NT * EPT
NPAD = 10240
RPT = NPAD // NS
OPT = 624
HR = NPAD // D
BR = 1000


NBUF = 3
NIB = 6


def _edge_ops(h, srcp3, dstp3, acc, rows, sidxb, didxb, sg, ss, si, base):
    def idx_issue(i, q):
        pltpu.async_copy(srcp3.at[base + i, 0], sidxb[q], si[q])
        pltpu.async_copy(dstp3.at[base + i, 0], didxb[q], si[q])

    def idx_wait(i, q):
        pltpu.make_async_copy(srcp3.at[base + i, 0], sidxb[q], si[q]).wait()
        pltpu.make_async_copy(dstp3.at[base + i, 0], didxb[q], si[q]).wait()

    def gather_issue(b, q):
        pltpu.async_copy(h.at[sidxb[q]], rows[b], sg[b])

    def gather_wait(b, q):
        pltpu.make_async_copy(h.at[sidxb[q]], rows[b], sg[b]).wait()

    def scatter_issue(b, q):
        pltpu.async_copy(rows[b], acc.at[didxb[q]], ss[b], add=True)

    def scatter_wait(b, q):
        pltpu.make_async_copy(rows[b], acc.at[didxb[q]], ss[b]).wait()

    return idx_issue, idx_wait, gather_issue, gather_wait, \
        scatter_issue, scatter_wait


def _run_edges(h, srcp3, dstp3, acc, rows, sidxb, didxb, sg, ss, si, base):
    idx_issue, idx_wait, gather_issue, gather_wait, scatter_issue, \
        scatter_wait = _edge_ops(h, srcp3, dstp3, acc, rows, sidxb, didxb,
                                 sg, ss, si, base)

    def slot(i, j, has_prev=True, has_g2=True, has_i4=True):
        b, q = j % NBUF, j % NIB
        gather_wait(b, q)
        scatter_issue(b, q)
        if has_prev:
            scatter_wait((j + 2) % NBUF, (j - 1) % NIB)
        if has_g2:
            idx_wait(i + 2, (j + 2) % NIB)
            gather_issue((j + 2) % NBUF, (j + 2) % NIB)
        if has_i4:
            idx_issue(i + 4, (j + 4) % NIB)

    for i in range(4):
        idx_issue(i, i)
    for i in range(2):
        idx_wait(i, i)
        gather_issue(i, i)

    for i in range(6):
        slot(i, i, has_prev=(i >= 1))

    def steady(g, carry):
        for j in range(6):
            slot(g * 6 + j, j)
        return carry

    lax.fori_loop(1, NITER // 6 - 1, steady, 0)

    for i in range(NITER - 6, NITER):
        slot(i, i % 6, has_g2=(i + 2 < NITER), has_i4=(i + 4 < NITER))
    scatter_wait((NITER - 1) % NBUF, (NITER - 1) % NIB)


def _sc_segsum_body(h, srcp3, dstp3, out0, out1, acc, *scratch):
    rows = list(scratch[0:3])
    sidxb = list(scratch[3:9])
    didxb = list(scratch[9:15])
    sg = list(scratch[15:18])
    ss = list(scratch[18:21])
    si = list(scratch[21:27])
    c = lax.axis_index("c")
    s = lax.axis_index("s")
    tile = c * NS + s

    with jax.named_scope("zero"):
        z16 = jnp.zeros((16,), jnp.float32)

        def zrow(i, carry):
            for k in range(D // 16):
                rows[0][i, pl.ds(k * 16, 16)] = z16
            return carry

        lax.fori_loop(0, CHUNK, zrow, 0)
        for k in range(RPT // CHUNK):
            pltpu.sync_copy(rows[0],
                            acc.at[pl.ds(s * RPT + k * CHUNK, CHUNK)])
        rem = RPT - (RPT // CHUNK) * CHUNK
        if rem:
            pltpu.sync_copy(
                rows[0].at[pl.ds(0, rem)],
                acc.at[pl.ds(s * RPT + RPT - rem, rem)])
        plsc.subcore_barrier()

    with jax.named_scope("edges"):
        _run_edges(h, srcp3, dstp3, acc, rows, sidxb, didxb, sg, ss, si,
                   tile * NITER)

    with jax.named_scope("waitall"):
        plsc.subcore_barrier()

    rows = pl.ds(s * OPT, OPT)
    tail = pl.ds(NS * OPT, N - NS * OPT)

    with jax.named_scope("writeout"):
        @pl.when(c == 0)
        def _():
            pltpu.sync_copy(acc.at[rows], out0.at[rows])

        @pl.when(c == 1)
        def _():
            pltpu.sync_copy(acc.at[rows], out1.at[rows])

        @pl.when((c == 0) & (s == NS - 1))
        def _():
            pltpu.sync_copy(acc.at[tail], out0.at[tail])

        @pl.when((c == 1) & (s == NS - 1))
        def _():
            pltpu.sync_copy(acc.at[tail], out1.at[tail])


_sc_segsum = pl.kernel(
    _sc_segsum_body,
    out_type=(jax.ShapeDtypeStruct((N, D), jnp.float32),
              jax.ShapeDtypeStruct((N, D), jnp.float32)),
    mesh=plsc.VectorSubcoreMesh(core_axis_name="c", subcore_axis_name="s"),
    scratch_types=(
        [pltpu.VMEM_SHARED((NPAD, D), jnp.float32)]
        + [pltpu.VMEM((CHUNK, D), jnp.float32)] * NBUF
        + [pltpu.VMEM((CHUNK,), jnp.int32)] * (2 * NIB)
        + [pltpu.SemaphoreType.DMA] * (2 * NBUF + NIB)
    ),
)


def _sc_degree_body(dstf, iota_hbm, out0, out1,
                    acc, hist, didx, idx80, sem):
    c = lax.axis_index("c")
    s = lax.axis_index("s")
    tile = c * NS + s

    cp = pltpu.async_copy(dstf.at[pl.ds(tile * EPT, EPT)], didx, sem)
    pltpu.sync_copy(iota_hbm, idx80)
    z16 = jnp.zeros((16,), jnp.float32)
    for r in range(HR):
        for k in range(D // 16):
            hist[r, pl.ds(k * 16, 16)] = z16

    @pl.when(s < HR // 8)
    def _():
        pltpu.sync_copy(hist.at[pl.ds(0, 8)], acc.at[pl.ds(s * 8, 8)])

    cp.wait()
    plsc.subcore_barrier()

    def step(g, carry):
        for k in range(6):
            d16 = didx[pl.ds(g * 96 + k * 16, 16)]
            cnt, last = plsc.scan_count(d16)
            plsc.addupdate_scatter(
                hist,
                [lax.shift_right_logical(d16, 7),
                 lax.bitwise_and(d16, 127)],
                cnt.astype(jnp.float32),
                mask=last,
            )
        return carry

    lax.fori_loop(0, EPT // 96, step, 0)
    pltpu.sync_copy(hist, acc.at[idx80], add=True)
    plsc.subcore_barrier()

    @pl.when((c == 0) & (s < HR // 8))
    def _():
        pltpu.sync_copy(acc.at[pl.ds(s * 8, 8)], out0.at[pl.ds(s * 8, 8)])

    @pl.when((c == 1) & (s < HR // 8))
    def _():
        pltpu.sync_copy(acc.at[pl.ds(s * 8, 8)], out1.at[pl.ds(s * 8, 8)])


_sc_degree = pl.kernel(
    _sc_degree_body,
    out_type=(jax.ShapeDtypeStruct((HR, D), jnp.float32),
              jax.ShapeDtypeStruct((HR, D), jnp.float32)),
    mesh=plsc.VectorSubcoreMesh(core_axis_name="c", subcore_axis_name="s"),
    scratch_types=[
        pltpu.VMEM_SHARED((HR, D), jnp.float32),
        pltpu.VMEM((HR, D), jnp.float32),
        pltpu.VMEM((EPT,), jnp.int32),
        pltpu.VMEM((HR,), jnp.int32),
        pltpu.SemaphoreType.DMA,
    ],
    compiler_params=pltpu.CompilerParams(needs_layout_passes=False),
)


def _conv(h, mean, wl_ref, wr_ref, bl_ref, br_ref):
    return (jnp.dot(mean, wl_ref[:, :], preferred_element_type=jnp.float32)
            + jnp.dot(h, wr_ref[:, :], preferred_element_type=jnp.float32)
            + bl_ref[:, :] + br_ref[:, :])


def _tc_layer_body(residual, hp_ref, p0_ref, p1_ref, inv_ref, wl_ref, wr_ref,
                   bl_ref, br_ref, o_ref):
    h = hp_ref[:, :]
    mean = (p0_ref[:, :] + p1_ref[:, :]) * inv_ref[:, :]
    z = _conv(h, mean, wl_ref, wr_ref, bl_ref, br_ref)
    if residual:
        z = z + h
        mu = jnp.mean(z, axis=1, keepdims=True)
        var = jnp.mean((z - mu) ** 2, axis=1, keepdims=True)
        z = (z - mu) * lax.rsqrt(var + 1e-5)
    o_ref[:, :] = jnp.maximum(z, 0.0)


def _tc_last_body(hp_ref, p0_ref, p1_ref, inv_ref, wl_ref, wr_ref,
                  bl_ref, br_ref, woutp_ref, boutp_ref, o_ref):
    h = hp_ref[:, :]
    mean = (p0_ref[:, :] + p1_ref[:, :]) * inv_ref[:, :]
    z = _conv(h, mean, wl_ref, wr_ref, bl_ref, br_ref)
    z = z + h
    mu = jnp.mean(z, axis=1, keepdims=True)
    var = jnp.mean((z - mu) ** 2, axis=1, keepdims=True)
    z = (z - mu) * lax.rsqrt(var + 1e-5)
    z = jnp.maximum(z, 0.0)
    logits = jnp.dot(z, woutp_ref[:, :], preferred_element_type=jnp.float32)
    logits = logits + boutp_ref[:, :]
    o_ref[:, :] = logits[:, :2]


_ROW = lambda i: (i, 0)
_FIX = lambda i: (0, 0)


def _tc_layer(residual, h, p0, p1, inv_cnt, Wl, Wr, bl, br):
    return pl.pallas_call(
        functools.partial(_tc_layer_body, residual),
        grid=(N // BR,),
        in_specs=[
            pl.BlockSpec((BR, D), _ROW),
            pl.BlockSpec((BR, D), _ROW),
            pl.BlockSpec((BR, D), _ROW),
            pl.BlockSpec((BR, 1), _ROW),
            pl.BlockSpec((D, D), _FIX),
            pl.BlockSpec((D, D), _FIX),
            pl.BlockSpec((1, D), _FIX),
            pl.BlockSpec((1, D), _FIX),
        ],
        out_specs=pl.BlockSpec((BR, D), _ROW),
        out_shape=jax.ShapeDtypeStruct((N, D), jnp.float32),
    )(h, p0, p1, inv_cnt, Wl, Wr, bl.reshape(1, D), br.reshape(1, D))


def _tc_last(h, p0, p1, inv_cnt, Wl, Wr, bl, br, Woutp, boutp):
    return pl.pallas_call(
        _tc_last_body,
        grid=(N // BR,),
        in_specs=[
            pl.BlockSpec((BR, D), _ROW),
            pl.BlockSpec((BR, D), _ROW),
            pl.BlockSpec((BR, D), _ROW),
            pl.BlockSpec((BR, 1), _ROW),
            pl.BlockSpec((D, D), _FIX),
            pl.BlockSpec((D, D), _FIX),
            pl.BlockSpec((1, D), _FIX),
            pl.BlockSpec((1, D), _FIX),
            pl.BlockSpec((D, D), _FIX),
            pl.BlockSpec((1, D), _FIX),
        ],
        out_specs=pl.BlockSpec((BR, 2), _ROW),
        out_shape=jax.ShapeDtypeStruct((N, 2), jnp.float32),
    )(h, p0, p1, inv_cnt, Wl, Wr, bl.reshape(1, D), br.reshape(1, D),
      Woutp, boutp)


def kernel(x, edge_index, Wl0, bl0, Wr0, br0, Wl1, bl1, Wr1, br1,
           Wl2, bl2, Wr2, br2, Wout, bout):
    f32 = jnp.float32
    x = x.astype(f32)
    src = edge_index[0].astype(jnp.int32)
    dst = edge_index[1].astype(jnp.int32)
    dummy_src = jnp.arange(EPAD - E, dtype=jnp.int32) % N
    srcp = jnp.concatenate([src, dummy_src]).reshape(-1, 1, CHUNK)
    dummy = N + jnp.arange(EPAD - E, dtype=jnp.int32) % (NPAD - N)
    dstf = jnp.concatenate([dst, dummy])
    dstp = dstf.reshape(-1, 1, CHUNK)
    iota80 = jnp.arange(HR, dtype=jnp.int32)
    Woutp = jnp.zeros((D, D), f32).at[:, :2].set(Wout.astype(f32))
    boutp = jnp.zeros((1, D), f32).at[:, :2].set(bout.astype(f32)[None, :])

    c0, c1 = _sc_degree(dstf, iota80)
    cnt = (c0 + c1).reshape(NPAD)[:N].reshape(N, 1)
    inv_cnt = 1.0 / jnp.maximum(cnt, 1.0)

    h = x
    p0, p1 = _sc_segsum(h, srcp, dstp)
    h = _tc_layer(False, h, p0, p1, inv_cnt, Wl0, Wr0, bl0, br0)
    p0, p1 = _sc_segsum(h, srcp, dstp)
    h = _tc_layer(True, h, p0, p1, inv_cnt, Wl1, Wr1, bl1, br1)
    p0, p1 = _sc_segsum(h, srcp, dstp)
    return _tc_last(h, p0, p1, inv_cnt, Wl2, Wr2, bl2, br2, Woutp, boutp)
